# baseline (speedup 1.0000x reference)
import functools

import jax
import jax.numpy as jnp
from jax import lax
from jax.experimental import pallas as pl
from jax.experimental.pallas import tpu as pltpu
from jax.experimental.pallas import tpu_sc as plsc

_VOCAB = 49408
_DIM = 1280
_NIDX = 16
_NW = 32
_CHUNK = 32
_NCHUNKS = _VOCAB // _CHUNK
_BASE_PER_W = _NCHUNKS // _NW
_EXTRA = _NCHUNKS - _BASE_PER_W * _NW


def _sc_body(emb_hbm, tok_hbm, idx_hbm, out_hbm, idx_v, tok_v, buf):
    c = lax.axis_index("c")
    s = lax.axis_index("s")
    wid = s * 2 + c

    pltpu.sync_copy(idx_hbm, idx_v)
    pltpu.sync_copy(tok_hbm, tok_v)
    iv = idx_v[...]
    idx_s = [iv[i] for i in range(_NIDX)]

    n_chunks = _BASE_PER_W + jnp.where(wid < _EXTRA, 1, 0).astype(jnp.int32)

    def chunk_body(j, carry):
        k = wid + j * _NW
        base = k * _CHUNK
        pltpu.sync_copy(emb_hbm.at[pl.ds(base, _CHUNK)], buf)
        for i in range(_NIDX):
            local = idx_s[i] - base

            @pl.when(
                jnp.logical_and(idx_s[i] >= base, idx_s[i] < base + _CHUNK)
            )
            def _(i=i, local=local):
                def _add(d, c2):
                    buf[local, pl.ds(d * 16, 16)] = (
                        buf[local, pl.ds(d * 16, 16)]
                        + tok_v[i, pl.ds(d * 16, 16)]
                    )
                    return c2

                lax.fori_loop(0, _DIM // 16, _add, 0)

        pltpu.sync_copy(buf, out_hbm.at[pl.ds(base, _CHUNK)])
        return carry

    lax.fori_loop(0, n_chunks, chunk_body, 0)


def kernel(embeddings, token_embeddings, indices):
    mesh = plsc.VectorSubcoreMesh(core_axis_name="c", subcore_axis_name="s")
    run = functools.partial(
        pl.kernel,
        out_type=jax.ShapeDtypeStruct((_VOCAB, _DIM), jnp.float32),
        mesh=mesh,
        scratch_types=[
            pltpu.VMEM((_NIDX,), jnp.int32),
            pltpu.VMEM((_NIDX, _DIM), jnp.float32),
            pltpu.VMEM((_CHUNK, _DIM), jnp.float32),
        ],
    )(_sc_body)
    return run(embeddings, token_embeddings, indices)


# speedup vs baseline: 1.1633x; 1.1633x over previous
import functools

import jax
import jax.numpy as jnp
from jax import lax
from jax.experimental import pallas as pl
from jax.experimental.pallas import tpu as pltpu
from jax.experimental.pallas import tpu_sc as plsc

_VOCAB = 49408
_DIM = 1280
_NIDX = 16
_NW = 32
_CHUNK = 32
_NCHUNKS = _VOCAB // _CHUNK
_NSTAGE = 2 * ((_NCHUNKS + 2 * _NW - 1) // (2 * _NW))


def _sc_body(
    emb_hbm, tok_hbm, idx_hbm, out_hbm,
    idx_v, tok_v, buf0, buf1, in0, in1, out0, out1,
):
    c = lax.axis_index("c")
    s = lax.axis_index("s")
    wid = s * 2 + c

    pltpu.sync_copy(idx_hbm, idx_v)
    pltpu.sync_copy(tok_hbm, tok_v)
    iv = idx_v[...]
    idx_s = [iv[i] for i in range(_NIDX)]

    def start_in(j, buf, sem):
        k = wid + j * _NW
        pltpu.async_copy(emb_hbm.at[pl.ds(k * _CHUNK, _CHUNK)], buf, sem)

    def stage(j, buf, in_sem, out_sem):
        k = wid + j * _NW

        @pl.when(k < _NCHUNKS)
        def _():
            base = k * _CHUNK
            pltpu.make_async_copy(
                emb_hbm.at[pl.ds(base, _CHUNK)], buf, in_sem
            ).wait()
            for i in range(_NIDX):
                local = idx_s[i] - base

                @pl.when(
                    jnp.logical_and(idx_s[i] >= base, idx_s[i] < base + _CHUNK)
                )
                def _(i=i, local=local):
                    def _add(d, c2):
                        buf[local, pl.ds(d * 16, 16)] = (
                            buf[local, pl.ds(d * 16, 16)]
                            + tok_v[i, pl.ds(d * 16, 16)]
                        )
                        return c2

                    lax.fori_loop(0, _DIM // 16, _add, 0)

            pltpu.async_copy(buf, out_hbm.at[pl.ds(base, _CHUNK)], out_sem)

        k2 = wid + (j + 2) * _NW

        @pl.when(k2 < _NCHUNKS)
        def _():
            pltpu.make_async_copy(
                buf, out_hbm.at[pl.ds(k * _CHUNK, _CHUNK)], out_sem
            ).wait()
            start_in(j + 2, buf, in_sem)

    start_in(0, buf0, in0)
    start_in(1, buf1, in1)

    def group(g, carry):
        stage(2 * g, buf0, in0, out0)
        stage(2 * g + 1, buf1, in1, out1)
        return carry

    lax.fori_loop(0, _NSTAGE // 2, group, 0)

    pltpu.make_async_copy(emb_hbm.at[pl.ds(0, _CHUNK)], buf0, out0).wait()
    pltpu.make_async_copy(emb_hbm.at[pl.ds(0, _CHUNK)], buf1, out1).wait()


def kernel(embeddings, token_embeddings, indices):
    mesh = plsc.VectorSubcoreMesh(core_axis_name="c", subcore_axis_name="s")
    run = functools.partial(
        pl.kernel,
        out_type=jax.ShapeDtypeStruct((_VOCAB, _DIM), jnp.float32),
        mesh=mesh,
        scratch_types=[
            pltpu.VMEM((_NIDX,), jnp.int32),
            pltpu.VMEM((_NIDX, _DIM), jnp.float32),
            pltpu.VMEM((_CHUNK, _DIM), jnp.float32),
            pltpu.VMEM((_CHUNK, _DIM), jnp.float32),
            pltpu.SemaphoreType.DMA,
            pltpu.SemaphoreType.DMA,
            pltpu.SemaphoreType.DMA,
            pltpu.SemaphoreType.DMA,
        ],
    )(_sc_body)
    return run(embeddings, token_embeddings, indices)


# speedup vs baseline: 1.2866x; 1.1060x over previous
import functools

import jax
import jax.numpy as jnp
from jax import lax
from jax.experimental import pallas as pl
from jax.experimental.pallas import tpu as pltpu
from jax.experimental.pallas import tpu_sc as plsc

_VOCAB = 49408
_DIM = 1280
_NIDX = 16
_BLOCK_ROWS = 2560
_NBLOCKS = (_VOCAB + _BLOCK_ROWS - 1) // _BLOCK_ROWS


def _sc_rows_body(emb_hbm, tok_hbm, idx_hbm, new_hbm, idx_v, row_v, tok_v, sem):
    c = lax.axis_index("c")
    s = lax.axis_index("s")
    wid = s * 2 + c

    @pl.when(wid < _NIDX)
    def _():
        pltpu.sync_copy(idx_hbm, idx_v)
        iv = idx_v[...]
        for i in range(_NIDX):
            @pl.when(wid == i)
            def _(i=i):
                idx_i = iv[i]
                pltpu.async_copy(emb_hbm.at[pl.ds(idx_i, 1)], row_v, sem).wait()
                pltpu.sync_copy(tok_hbm.at[pl.ds(i, 1)], tok_v)

                def _add(d, carry):
                    row_v[0, pl.ds(d * 16, 16)] = (
                        row_v[0, pl.ds(d * 16, 16)] + tok_v[0, pl.ds(d * 16, 16)]
                    )
                    return carry

                lax.fori_loop(0, _DIM // 16, _add, 0)
                pltpu.sync_copy(row_v, new_hbm.at[pl.ds(i, 1)])


def _sc_new_rows(embeddings, token_embeddings, indices):
    mesh = plsc.VectorSubcoreMesh(core_axis_name="c", subcore_axis_name="s")
    run = functools.partial(
        pl.kernel,
        out_type=jax.ShapeDtypeStruct((_NIDX, _DIM), jnp.float32),
        mesh=mesh,
        scratch_types=[
            pltpu.VMEM((_NIDX,), jnp.int32),
            pltpu.VMEM((1, _DIM), jnp.float32),
            pltpu.VMEM((1, _DIM), jnp.float32),
            pltpu.SemaphoreType.DMA,
        ],
    )(_sc_rows_body)
    return run(embeddings, token_embeddings, indices)


def _tc_clone_body(in_ref, out_ref):
    out_ref[...] = in_ref[...]


def _tc_clone(embeddings):
    return pl.pallas_call(
        _tc_clone_body,
        grid=(_NBLOCKS,),
        in_specs=[pl.BlockSpec((_BLOCK_ROWS, _DIM), lambda i: (i, 0))],
        out_specs=pl.BlockSpec((_BLOCK_ROWS, _DIM), lambda i: (i, 0)),
        out_shape=jax.ShapeDtypeStruct((_VOCAB, _DIM), jnp.float32),
    )(embeddings)


def _tc_patch_body(idx_ref, new_ref, alias_ref, out_ref, sem):
    del alias_ref
    for i in range(_NIDX):
        idx = idx_ref[i]
        pltpu.make_async_copy(
            new_ref.at[pl.ds(i, 1)], out_ref.at[pl.ds(idx, 1)], sem
        ).start()
        pltpu.make_async_copy(
            new_ref.at[pl.ds(i, 1)], out_ref.at[pl.ds(idx, 1)], sem
        ).wait()


def _tc_patch(cloned, new_rows, indices):
    return pl.pallas_call(
        _tc_patch_body,
        in_specs=[
            pl.BlockSpec(memory_space=pltpu.SMEM),
            pl.BlockSpec(memory_space=pltpu.VMEM),
            pl.BlockSpec(memory_space=pl.ANY),
        ],
        out_specs=pl.BlockSpec(memory_space=pl.ANY),
        out_shape=jax.ShapeDtypeStruct((_VOCAB, _DIM), jnp.float32),
        input_output_aliases={2: 0},
        scratch_shapes=[pltpu.SemaphoreType.DMA],
    )(indices, new_rows, cloned)


def kernel(embeddings, token_embeddings, indices):
    new_rows = _sc_new_rows(embeddings, token_embeddings, indices)
    cloned = _tc_clone(embeddings)
    return _tc_patch(cloned, new_rows, indices)


# speedup vs baseline: 1.5018x; 1.1673x over previous
import jax
import jax.numpy as jnp
from jax.experimental import pallas as pl
from jax.experimental.pallas import tpu as pltpu

_VOCAB = 49408
_DIM = 1280
_NIDX = 16
_BLOCK_ROWS = 2816
_NBLOCKS = (_VOCAB + _BLOCK_ROWS - 1) // _BLOCK_ROWS


def _body(idx_ref, in_ref, tok_ref, out_ref):
    out_ref[...] = in_ref[...]
    base = pl.program_id(0) * _BLOCK_ROWS
    for i in range(_NIDX):
        idx = idx_ref[i]
        local = idx - base

        @pl.when(jnp.logical_and(idx >= base, idx < base + _BLOCK_ROWS))
        def _():
            out_ref[pl.ds(local, 1), :] = (
                out_ref[pl.ds(local, 1), :] + tok_ref[pl.ds(i, 1), :]
            )


def kernel(embeddings, token_embeddings, indices):
    return pl.pallas_call(
        _body,
        grid=(_NBLOCKS,),
        in_specs=[
            pl.BlockSpec(memory_space=pltpu.SMEM),
            pl.BlockSpec((_BLOCK_ROWS, _DIM), lambda i: (i, 0)),
            pl.BlockSpec((_NIDX, _DIM), lambda i: (0, 0)),
        ],
        out_specs=pl.BlockSpec((_BLOCK_ROWS, _DIM), lambda i: (i, 0)),
        out_shape=jax.ShapeDtypeStruct((_VOCAB, _DIM), jnp.float32),
    )(indices, embeddings, token_embeddings)
